# baseline (speedup 1.0000x reference)
import functools

import jax
import jax.numpy as jnp
from jax import lax
from jax.experimental import pallas as pl
from jax.experimental.pallas import tpu as pltpu
from jax.experimental.pallas import tpu_sc as plsc

N = 10000
NPAD = 10112
D = 256
H = 128
NS = 16
CHUNK = 128
NB = 2
NPHASE = 2
ROWS_PER_TILE = NPAD // NS


def _sc_aggregate(x2, zeros_tile, src5, dst4, nch):
    mesh = plsc.VectorSubcoreMesh(core_axis_name="c", subcore_axis_name="s")

    hc = nch // NPHASE

    @functools.partial(
        pl.kernel,
        out_type=jax.ShapeDtypeStruct((2, NPAD, H), jnp.float32),
        mesh=mesh,
        scratch_types=[
            pltpu.VMEM((hc, CHUNK), jnp.int32),
            pltpu.VMEM((hc, CHUNK), jnp.int32),
            pltpu.VMEM((NB, CHUNK, H), jnp.float32),
            pltpu.VMEM_SHARED((NPAD, H), jnp.float32),
            [pltpu.SemaphoreType.DMA] * NB,
        ],
    )
    def agg_kernel(x2_hbm, zeros_hbm, src_hbm, dst_hbm, out_hbm,
                   src_v, dst_v, rows_v, acc_sh, gsems):
        c = lax.axis_index("c")
        s = lax.axis_index("s")

        pltpu.sync_copy(src_hbm.at[c, s, 0], src_v)
        pltpu.sync_copy(dst_hbm.at[s, 0], dst_v)
        for b in range(NB):
            pltpu.async_copy(x2_hbm.at[src_v.at[b]], rows_v.at[b], gsems[b])
        sl = pl.ds(s * ROWS_PER_TILE, ROWS_PER_TILE)
        pltpu.sync_copy(zeros_hbm, acc_sh.at[sl])
        plsc.subcore_barrier()

        for p in range(NPHASE):
            if p > 0:
                pltpu.sync_copy(src_hbm.at[c, s, p], src_v)
                pltpu.sync_copy(dst_hbm.at[s, p], dst_v)
                for b in range(NB):
                    pltpu.async_copy(x2_hbm.at[src_v.at[b]], rows_v.at[b],
                                     gsems[b])

            def group(g, carry):
                base = g * NB
                for b in range(NB):
                    j = base + b
                    pltpu.make_async_copy(
                        x2_hbm.at[src_v.at[j]], rows_v.at[b], gsems[b]).wait()
                    pltpu.sync_copy(rows_v.at[b], acc_sh.at[dst_v.at[j]],
                                    add=True)

                    @pl.when(j + NB < hc)
                    def _():
                        pltpu.async_copy(x2_hbm.at[src_v.at[j + NB]],
                                         rows_v.at[b], gsems[b])
                return carry

            lax.fori_loop(0, hc // NB, group, 0)
        plsc.subcore_barrier()
        pltpu.sync_copy(acc_sh.at[sl], out_hbm.at[c, sl])

    return agg_kernel(x2, zeros_tile, src5, dst4)


def _tc_mlp(x, agg2, W1, b1, W2, b2, eps, gamma, beta, mean, var):
    BLK = 1000
    grid = (N // BLK,)

    def body(eps_r, x_r, lo_r, hi_r, W1_r, b1_r, W2_r, b2_r,
             g_r, be_r, mu_r, va_r, o_r):
        pre = jnp.concatenate([lo_r[0], hi_r[0]], axis=1)
        h = pre + (1.0 + eps_r[0]) * x_r[...]
        h = jnp.dot(h, W1_r[...], preferred_element_type=jnp.float32) + b1_r[...]
        h = jnp.maximum(h, 0.0)
        h = jnp.dot(h, W2_r[...], preferred_element_type=jnp.float32) + b2_r[...]
        scale = g_r[...] * lax.rsqrt(va_r[...] + 1e-5)
        o_r[...] = jnp.maximum((h - mu_r[...]) * scale + be_r[...], 0.0)

    row = lambda i: (i, 0)
    fixed = lambda i: (0, 0)
    return pl.pallas_call(
        body,
        grid=grid,
        in_specs=[
            pl.BlockSpec(memory_space=pltpu.SMEM),
            pl.BlockSpec((BLK, D), row),
            pl.BlockSpec((1, BLK, H), lambda i: (0, i, 0)),
            pl.BlockSpec((1, BLK, H), lambda i: (1, i, 0)),
            pl.BlockSpec((D, D), fixed),
            pl.BlockSpec((1, D), fixed),
            pl.BlockSpec((D, D), fixed),
            pl.BlockSpec((1, D), fixed),
            pl.BlockSpec((1, D), fixed),
            pl.BlockSpec((1, D), fixed),
            pl.BlockSpec((1, D), fixed),
            pl.BlockSpec((1, D), fixed),
        ],
        out_specs=pl.BlockSpec((BLK, D), row),
        out_shape=jax.ShapeDtypeStruct((N, D), jnp.float32),
    )(eps.reshape(1), x, agg2, agg2, W1, b1.reshape(1, D),
      W2, b2.reshape(1, D), gamma.reshape(1, D), beta.reshape(1, D),
      mean.reshape(1, D), var.reshape(1, D))


def kernel(x, edge_index, W1, b1, W2, b2, eps, gamma, beta,
           running_mean, running_var):
    E = edge_index.shape[1]
    epad = -E % (NS * CHUNK * NPHASE * NB)
    src = edge_index[0]
    dst = edge_index[1]
    if epad:
        src = jnp.concatenate([src, jnp.zeros((epad,), jnp.int32)])
        dst = jnp.concatenate([dst, jnp.full((epad,), N, jnp.int32)])
    nch = (E + epad) // (NS * CHUNK)
    hc = nch // NPHASE
    src2 = 2 * src
    src5 = jnp.stack([src2, src2 + 1]).reshape(2, NS, NPHASE, hc, CHUNK)
    dst4 = dst.reshape(NS, NPHASE, hc, CHUNK)
    x2 = x.reshape(2 * N, H)
    zeros_tile = jnp.zeros((ROWS_PER_TILE, H), jnp.float32)
    agg2 = _sc_aggregate(x2, zeros_tile, src5, dst4, nch)
    return _tc_mlp(x, agg2, W1, b1, W2, b2, eps,
                   gamma, beta, running_mean, running_var)


# speedup vs baseline: 1.0511x; 1.0511x over previous
import functools

import jax
import jax.numpy as jnp
from jax import lax
from jax.experimental import pallas as pl
from jax.experimental.pallas import tpu as pltpu
from jax.experimental.pallas import tpu_sc as plsc

N = 10000
NPAD = 10112
D = 256
H = 128
NS = 16
CHUNK = 128
NB = 2
NPHASE = 2
ROWS_PER_TILE = NPAD // NS


def _sc_aggregate(x_lo, x_hi, zeros_tile, src4, dst4, nch):
    mesh = plsc.VectorSubcoreMesh(core_axis_name="c", subcore_axis_name="s")

    hc = nch // NPHASE

    @functools.partial(
        pl.kernel,
        out_type=jax.ShapeDtypeStruct((2, NPAD, H), jnp.float32),
        mesh=mesh,
        scratch_types=[
            pltpu.VMEM((hc, CHUNK), jnp.int32),
            pltpu.VMEM((hc, CHUNK), jnp.int32),
            pltpu.VMEM((NB, CHUNK, H), jnp.float32),
            pltpu.VMEM_SHARED((NPAD, H), jnp.float32),
            [pltpu.SemaphoreType.DMA] * NB,
        ],
    )
    def agg_kernel(xlo_hbm, xhi_hbm, zeros_hbm, src_hbm, dst_hbm, out_hbm,
                   src_v, dst_v, rows_v, acc_sh, gsems):
        c = lax.axis_index("c")
        s = lax.axis_index("s")

        def run(xh):
            pltpu.sync_copy(src_hbm.at[s, 0], src_v)
            pltpu.sync_copy(dst_hbm.at[s, 0], dst_v)
            for b in range(NB):
                pltpu.async_copy(xh.at[src_v.at[b]], rows_v.at[b], gsems[b])
            sl = pl.ds(s * ROWS_PER_TILE, ROWS_PER_TILE)
            pltpu.sync_copy(zeros_hbm, acc_sh.at[sl])
            plsc.subcore_barrier()

            for p in range(NPHASE):
                if p > 0:
                    pltpu.sync_copy(src_hbm.at[s, p], src_v)
                    pltpu.sync_copy(dst_hbm.at[s, p], dst_v)
                    for b in range(NB):
                        pltpu.async_copy(xh.at[src_v.at[b]], rows_v.at[b],
                                         gsems[b])

                def group(g, carry):
                    base = g * NB
                    for b in range(NB):
                        j = base + b
                        pltpu.make_async_copy(
                            xh.at[src_v.at[j]], rows_v.at[b], gsems[b]).wait()
                        pltpu.sync_copy(rows_v.at[b], acc_sh.at[dst_v.at[j]],
                                        add=True)

                        @pl.when(j + NB < hc)
                        def _():
                            pltpu.async_copy(xh.at[src_v.at[j + NB]],
                                             rows_v.at[b], gsems[b])
                    return carry

                lax.fori_loop(0, hc // NB, group, 0)
            plsc.subcore_barrier()
            pltpu.sync_copy(acc_sh.at[sl], out_hbm.at[c, sl])

        @pl.when(c == 0)
        def _():
            run(xlo_hbm)

        @pl.when(c == 1)
        def _():
            run(xhi_hbm)

    return agg_kernel(x_lo, x_hi, zeros_tile, src4, dst4)


def _tc_mlp(x, agg2, W1, b1, W2, b2, eps, gamma, beta, mean, var):
    BLK = 1000
    grid = (N // BLK,)

    def body(eps_r, x_r, lo_r, hi_r, W1_r, b1_r, W2_r, b2_r,
             g_r, be_r, mu_r, va_r, o_r):
        pre = jnp.concatenate([lo_r[0], hi_r[0]], axis=1)
        h = pre + (1.0 + eps_r[0]) * x_r[...]
        h = jnp.dot(h, W1_r[...], preferred_element_type=jnp.float32) + b1_r[...]
        h = jnp.maximum(h, 0.0)
        h = jnp.dot(h, W2_r[...], preferred_element_type=jnp.float32) + b2_r[...]
        scale = g_r[...] * lax.rsqrt(va_r[...] + 1e-5)
        o_r[...] = jnp.maximum((h - mu_r[...]) * scale + be_r[...], 0.0)

    row = lambda i: (i, 0)
    fixed = lambda i: (0, 0)
    return pl.pallas_call(
        body,
        grid=grid,
        in_specs=[
            pl.BlockSpec(memory_space=pltpu.SMEM),
            pl.BlockSpec((BLK, D), row),
            pl.BlockSpec((1, BLK, H), lambda i: (0, i, 0)),
            pl.BlockSpec((1, BLK, H), lambda i: (1, i, 0)),
            pl.BlockSpec((D, D), fixed),
            pl.BlockSpec((1, D), fixed),
            pl.BlockSpec((D, D), fixed),
            pl.BlockSpec((1, D), fixed),
            pl.BlockSpec((1, D), fixed),
            pl.BlockSpec((1, D), fixed),
            pl.BlockSpec((1, D), fixed),
            pl.BlockSpec((1, D), fixed),
        ],
        out_specs=pl.BlockSpec((BLK, D), row),
        out_shape=jax.ShapeDtypeStruct((N, D), jnp.float32),
    )(eps.reshape(1), x, agg2, agg2, W1, b1.reshape(1, D),
      W2, b2.reshape(1, D), gamma.reshape(1, D), beta.reshape(1, D),
      mean.reshape(1, D), var.reshape(1, D))


def kernel(x, edge_index, W1, b1, W2, b2, eps, gamma, beta,
           running_mean, running_var):
    E = edge_index.shape[1]
    epad = -E % (NS * CHUNK * NPHASE * NB)
    src = edge_index[0]
    dst = edge_index[1]
    if epad:
        src = jnp.concatenate([src, jnp.zeros((epad,), jnp.int32)])
        dst = jnp.concatenate([dst, jnp.full((epad,), N, jnp.int32)])
    nch = (E + epad) // (NS * CHUNK)
    hc = nch // NPHASE
    src4 = src.reshape(NS, NPHASE, hc, CHUNK)
    dst4 = dst.reshape(NS, NPHASE, hc, CHUNK)
    x_lo = x[:, :H]
    x_hi = x[:, H:]
    zeros_tile = jnp.zeros((ROWS_PER_TILE, H), jnp.float32)
    agg2 = _sc_aggregate(x_lo, x_hi, zeros_tile, src4, dst4, nch)
    return _tc_mlp(x, agg2, W1, b1, W2, b2, eps,
                   gamma, beta, running_mean, running_var)


# speedup vs baseline: 1.0597x; 1.0082x over previous
import functools

import jax
import jax.numpy as jnp
from jax import lax
from jax.experimental import pallas as pl
from jax.experimental.pallas import tpu as pltpu
from jax.experimental.pallas import tpu_sc as plsc

N = 10000
NPAD = 10112
D = 256
H = 128
NS = 16
CHUNK = 128
NB = 2
NPHASE = 2
ROWS_PER_TILE = NPAD // NS


def _sc_aggregate(x_lo, x_hi, zeros_tile, src4, dst4, nch):
    mesh = plsc.VectorSubcoreMesh(core_axis_name="c", subcore_axis_name="s")

    hc = nch // NPHASE

    @functools.partial(
        pl.kernel,
        out_type=jax.ShapeDtypeStruct((2, NPAD, H), jnp.float32),
        mesh=mesh,
        compiler_params=pltpu.CompilerParams(use_tc_tiling_on_sc=False),
        scratch_types=[
            pltpu.VMEM((hc, CHUNK), jnp.int32),
            pltpu.VMEM((hc, CHUNK), jnp.int32),
            pltpu.VMEM((NB, CHUNK, H), jnp.float32),
            pltpu.VMEM_SHARED((NPAD, H), jnp.float32),
            [pltpu.SemaphoreType.DMA] * NB,
        ],
    )
    def agg_kernel(xlo_hbm, xhi_hbm, zeros_hbm, src_hbm, dst_hbm, out_hbm,
                   src_v, dst_v, rows_v, acc_sh, gsems):
        c = lax.axis_index("c")
        s = lax.axis_index("s")

        def run(xh):
            pltpu.sync_copy(src_hbm.at[s, 0], src_v)
            pltpu.sync_copy(dst_hbm.at[s, 0], dst_v)
            for b in range(NB):
                pltpu.async_copy(xh.at[src_v.at[b]], rows_v.at[b], gsems[b])
            sl = pl.ds(s * ROWS_PER_TILE, ROWS_PER_TILE)
            pltpu.sync_copy(zeros_hbm, acc_sh.at[sl])
            plsc.subcore_barrier()

            for p in range(NPHASE):
                if p > 0:
                    pltpu.sync_copy(src_hbm.at[s, p], src_v)
                    pltpu.sync_copy(dst_hbm.at[s, p], dst_v)
                    for b in range(NB):
                        pltpu.async_copy(xh.at[src_v.at[b]], rows_v.at[b],
                                         gsems[b])

                def group(g, carry):
                    base = g * NB
                    for b in range(NB):
                        j = base + b
                        pltpu.make_async_copy(
                            xh.at[src_v.at[j]], rows_v.at[b], gsems[b]).wait()
                        pltpu.sync_copy(rows_v.at[b], acc_sh.at[dst_v.at[j]],
                                        add=True)

                        @pl.when(j + NB < hc)
                        def _():
                            pltpu.async_copy(xh.at[src_v.at[j + NB]],
                                             rows_v.at[b], gsems[b])
                    return carry

                lax.fori_loop(0, hc // NB, group, 0)
            plsc.subcore_barrier()
            pltpu.sync_copy(acc_sh.at[sl], out_hbm.at[c, sl])

        @pl.when(c == 0)
        def _():
            run(xlo_hbm)

        @pl.when(c == 1)
        def _():
            run(xhi_hbm)

    return agg_kernel(x_lo, x_hi, zeros_tile, src4, dst4)


def _tc_mlp(x, agg2, W1, b1, W2, b2, eps, gamma, beta, mean, var):
    BLK = 1000
    grid = (N // BLK,)

    def body(eps_r, x_r, lo_r, hi_r, W1_r, b1_r, W2_r, b2_r,
             g_r, be_r, mu_r, va_r, o_r):
        pre = jnp.concatenate([lo_r[0], hi_r[0]], axis=1)
        h = pre + (1.0 + eps_r[0]) * x_r[...]
        h = jnp.dot(h, W1_r[...], preferred_element_type=jnp.float32) + b1_r[...]
        h = jnp.maximum(h, 0.0)
        h = jnp.dot(h, W2_r[...], preferred_element_type=jnp.float32) + b2_r[...]
        scale = g_r[...] * lax.rsqrt(va_r[...] + 1e-5)
        o_r[...] = jnp.maximum((h - mu_r[...]) * scale + be_r[...], 0.0)

    row = lambda i: (i, 0)
    fixed = lambda i: (0, 0)
    return pl.pallas_call(
        body,
        grid=grid,
        in_specs=[
            pl.BlockSpec(memory_space=pltpu.SMEM),
            pl.BlockSpec((BLK, D), row),
            pl.BlockSpec((1, BLK, H), lambda i: (0, i, 0)),
            pl.BlockSpec((1, BLK, H), lambda i: (1, i, 0)),
            pl.BlockSpec((D, D), fixed),
            pl.BlockSpec((1, D), fixed),
            pl.BlockSpec((D, D), fixed),
            pl.BlockSpec((1, D), fixed),
            pl.BlockSpec((1, D), fixed),
            pl.BlockSpec((1, D), fixed),
            pl.BlockSpec((1, D), fixed),
            pl.BlockSpec((1, D), fixed),
        ],
        out_specs=pl.BlockSpec((BLK, D), row),
        out_shape=jax.ShapeDtypeStruct((N, D), jnp.float32),
    )(eps.reshape(1), x, agg2, agg2, W1, b1.reshape(1, D),
      W2, b2.reshape(1, D), gamma.reshape(1, D), beta.reshape(1, D),
      mean.reshape(1, D), var.reshape(1, D))


def kernel(x, edge_index, W1, b1, W2, b2, eps, gamma, beta,
           running_mean, running_var):
    E = edge_index.shape[1]
    epad = -E % (NS * CHUNK * NPHASE * NB)
    src = edge_index[0]
    dst = edge_index[1]
    if epad:
        src = jnp.concatenate([src, jnp.zeros((epad,), jnp.int32)])
        dst = jnp.concatenate([dst, jnp.full((epad,), N, jnp.int32)])
    nch = (E + epad) // (NS * CHUNK)
    hc = nch // NPHASE
    src4 = src.reshape(NS, NPHASE, hc, CHUNK)
    dst4 = dst.reshape(NS, NPHASE, hc, CHUNK)
    x_lo = x[:, :H]
    x_hi = x[:, H:]
    zeros_tile = jnp.zeros((ROWS_PER_TILE, H), jnp.float32)
    agg2 = _sc_aggregate(x_lo, x_hi, zeros_tile, src4, dst4, nch)
    return _tc_mlp(x, agg2, W1, b1, W2, b2, eps,
                   gamma, beta, running_mean, running_var)
